# baseline (speedup 1.0000x reference)
import functools

import jax
import jax.numpy as jnp
import numpy as np
from jax import lax
from jax.experimental import pallas as pl
from jax.experimental.pallas import tpu as pltpu
from jax.experimental.pallas import tpu_sc as plsc

_N = 10000
_E = 320000
_D = 128
_H = 128
_B = 64
_OUT = 40
_L = 3

_NC = 2
_NS = 16
_NW = _NC * _NS
_CHUNK = 64
_NCHUNK = 160
_HC = 20
_NBUF = 5
_EPAD = _NW * _NCHUNK * _CHUNK
_NPAD = 10240
_RPT = _NPAD // _NS


_PAD_LANES = np.arange(_EPAD - _E, dtype=np.int32) % 128
_PAD_BLOCK = np.stack([_PAD_LANES * (_N // 128), _N + _PAD_LANES])


def _seg_sum_body(h_hbm, edges_hbm, out_hbm,
                  accum, idx_s, idx_d, rows, gsem, ssem):
    c = lax.axis_index("c")
    s = lax.axis_index("s")
    wid = c * _NS + s

    @pl.loop(0, _CHUNK)
    def _zero_rows(r):
        @pl.loop(0, _D // 16)
        def _zero_lanes(k):
            rows[0, r, pl.ds(k * 16, 16)] = jnp.zeros((16,), jnp.float32)

    for k in range(_RPT // _CHUNK):
        pltpu.sync_copy(rows.at[0],
                        accum.at[pl.ds(s * _RPT + k * _CHUNK, _CHUNK)])
    plsc.subcore_barrier()

    def g_start(j, b):
        pltpu.async_copy(h_hbm.at[idx_s.at[j]], rows.at[b], gsem.at[b])

    def g_wait(j, b):
        pltpu.make_async_copy(h_hbm.at[idx_s.at[j]], rows.at[b],
                              gsem.at[b]).wait()

    def s_start(j, b):
        pltpu.async_copy(rows.at[b], accum.at[idx_d.at[j]], ssem.at[b],
                         add=True)

    def s_wait(j, b):
        pltpu.make_async_copy(rows.at[b], accum.at[idx_d.at[j]],
                              ssem.at[b]).wait()

    for stage in range(_NCHUNK // _HC):
        pltpu.sync_copy(edges_hbm.at[0].at[wid].at[stage], idx_s)
        pltpu.sync_copy(edges_hbm.at[1].at[wid].at[stage], idx_d)

        for j in (0, 1, 2):
            g_start(j, j)
        for j in (0, 1):
            g_start(j + 3, j + 3)
            g_wait(j, j)
            s_start(j, j)

        @pl.loop(0, (_HC - 5) // _NBUF)
        def _edge_loop(k):
            for u in range(_NBUF):
                j = 2 + k * _NBUF + u
                b = (2 + u) % _NBUF
                b3 = (2 + u + 3) % _NBUF
                s_wait(j - 2, b3)
                g_start(j + 3, b3)
                g_wait(j, b)
                s_start(j, b)

        for j in range(_HC - 3, _HC):
            s_wait(j - 2, (j - 2) % _NBUF)
            g_wait(j, j % _NBUF)
            s_start(j, j % _NBUF)
        for j in (_HC - 2, _HC - 1):
            s_wait(j, j % _NBUF)

    plsc.subcore_barrier()
    pltpu.sync_copy(accum.at[pl.ds(s * _RPT, _RPT)],
                    out_hbm.at[c].at[pl.ds(s * _RPT, _RPT)])


_segment_sum_sc = functools.partial(
    pl.kernel,
    out_type=jax.ShapeDtypeStruct((_NC, _NPAD, _D), jnp.float32),
    mesh=plsc.VectorSubcoreMesh(core_axis_name="c", subcore_axis_name="s"),
    scratch_types=[
        pltpu.VMEM_SHARED((_NPAD, _D), jnp.float32),
        pltpu.VMEM((_HC, _CHUNK), jnp.int32),
        pltpu.VMEM((_HC, _CHUNK), jnp.int32),
        pltpu.VMEM((_NBUF, _CHUNK, _D), jnp.float32),
        pltpu.SemaphoreType.DMA((_NBUF,)),
        pltpu.SemaphoreType.DMA((_NBUF,)),
    ],
)(_seg_sum_body)


def _bn_relu(z, g, b):
    m = jnp.mean(z, axis=0, keepdims=True)
    v = jnp.mean((z - m) ** 2, axis=0, keepdims=True)
    return jnp.maximum((z - m) / jnp.sqrt(v + 1e-5) * g + b, 0.0)


def _dot_bf16(a, b):
    return jnp.dot(a.astype(jnp.bfloat16), b.astype(jnp.bfloat16),
                   preferred_element_type=jnp.float32)


_BLK = 1000
_NB = _N // _BLK


def _phase_a(i, h_ref, agg_ref, eps_ref, w1_ref, b1_ref, z_acc, s1, s2):
    @pl.when(i == 0)
    def _init():
        s1[...] = jnp.zeros_like(s1)
        s2[...] = jnp.zeros_like(s2)

    @pl.when(i < _NB)
    def _stream():
        y = (1.0 + eps_ref[0, 0]) * h_ref[...] + agg_ref[0] + agg_ref[1]
        z = _dot_bf16(y, w1_ref[...]) + b1_ref[...]
        z_acc[pl.ds(i * _BLK, _BLK), :] = z
        s1[...] += jnp.sum(z, axis=0, keepdims=True)
        s2[...] += jnp.sum(z * z, axis=0, keepdims=True)


def _phase_b(g1_ref, be1_ref, w2_ref, b2_ref, g_ref, be_ref, z_acc, s1, s2):
    m = s1[...] / _N
    v = s2[...] / _N - m * m
    z = jnp.maximum((z_acc[...] - m) / jnp.sqrt(v + 1e-5) * g1_ref[...]
                    + be1_ref[...], 0.0)
    a = _dot_bf16(z, w2_ref[...]) + b2_ref[...]
    return _bn_relu(a, g_ref[...], be_ref[...])


def _gin_layer_body(h_ref, agg_ref, eps_ref, w1_ref, b1_ref, g1_ref, be1_ref,
                    w2_ref, b2_ref, g_ref, be_ref, out_ref, z_acc, s1, s2):
    i = pl.program_id(0)
    _phase_a(i, h_ref, agg_ref, eps_ref, w1_ref, b1_ref, z_acc, s1, s2)

    @pl.when(i == _NB)
    def _final():
        out_ref[...] = _phase_b(g1_ref, be1_ref, w2_ref, b2_ref, g_ref,
                                be_ref, z_acc, s1, s2)


def _final_body(h_ref, agg_ref, batch_ref, eps_ref, w1_ref, b1_ref, g1_ref,
                be1_ref, w2_ref, b2_ref, g_ref, be_ref, l1w_ref, l1b_ref,
                bng_ref, bnb_ref, l2w_ref, l2b_ref, out_ref, z_acc, s1, s2):
    i = pl.program_id(0)
    _phase_a(i, h_ref, agg_ref, eps_ref, w1_ref, b1_ref, z_acc, s1, s2)

    @pl.when(i == _NB)
    def _final():
        hl = _phase_b(g1_ref, be1_ref, w2_ref, b2_ref, g_ref, be_ref,
                      z_acc, s1, s2)
        seg_ids = lax.broadcasted_iota(jnp.int32, (_B, _N), 0)
        onehot = (seg_ids == batch_ref[...]).astype(jnp.float32)
        pooled = _dot_bf16(onehot, hl)
        z = jnp.dot(pooled, l1w_ref[...], preferred_element_type=jnp.float32)
        z = _bn_relu(z + l1b_ref[...], bng_ref[...], bnb_ref[...])
        z = jnp.dot(z, l2w_ref[...],
                    preferred_element_type=jnp.float32) + l2b_ref[...]
        zm = z - jnp.max(z, axis=1, keepdims=True)
        out_ref[...] = zm - jnp.log(jnp.sum(jnp.exp(zm), axis=1,
                                            keepdims=True))


def _row_block(i):
    return (jnp.minimum(i, _NB - 1), 0)


_LAYER_SPECS = [
    pl.BlockSpec((_BLK, _D), _row_block),
    pl.BlockSpec((2, _BLK, _D), lambda i: (0,) + _row_block(i)),
    pl.BlockSpec(memory_space=pltpu.SMEM),
] + [pl.BlockSpec(memory_space=pltpu.VMEM)] * 8

_SCRATCH = [
    pltpu.VMEM((_N, 2 * _H), jnp.float32),
    pltpu.VMEM((1, 2 * _H), jnp.float32),
    pltpu.VMEM((1, 2 * _H), jnp.float32),
]


def _gin_layer(h, agg, eps, w1, b1, g1, be1, w2, b2, g, be):
    return pl.pallas_call(
        _gin_layer_body,
        grid=(_NB + 1,),
        out_shape=jax.ShapeDtypeStruct((_N, _H), jnp.float32),
        in_specs=_LAYER_SPECS,
        out_specs=pl.BlockSpec((_N, _H), lambda i: (0, 0)),
        scratch_shapes=_SCRATCH,
    )(h, agg, eps, w1, b1, g1, be1, w2, b2, g, be)


def _final_layer(h, agg, batch2d, eps, w1, b1, g1, be1, w2, b2, g, be,
                 l1w, l1b, bng, bnb, l2w, l2b):
    in_specs = (_LAYER_SPECS[:2]
                + [pl.BlockSpec(memory_space=pltpu.VMEM)]
                + _LAYER_SPECS[2:]
                + [pl.BlockSpec(memory_space=pltpu.VMEM)] * 6)
    return pl.pallas_call(
        _final_body,
        grid=(_NB + 1,),
        out_shape=jax.ShapeDtypeStruct((_B, _OUT), jnp.float32),
        in_specs=in_specs,
        out_specs=pl.BlockSpec((_B, _OUT), lambda i: (0, 0)),
        scratch_shapes=_SCRATCH,
    )(h, agg, batch2d, eps, w1, b1, g1, be1, w2, b2, g, be,
      l1w, l1b, bng, bnb, l2w, l2b)


def kernel(x, edge_index, batch, params):
    edges = jnp.concatenate(
        [edge_index, jnp.asarray(_PAD_BLOCK)], axis=1,
    ).reshape(2, _NW, _NCHUNK // _HC, _HC, _CHUNK)
    batch2d = batch.reshape(1, _N)

    def vec(name):
        p = params[name]
        return p.reshape(1, p.shape[0])

    h = x
    out = None
    for i in range(_L):
        agg = _segment_sum_sc(h, edges)
        eps = params[f"eps_{i}"].reshape(1, 1)
        layer_args = (eps, params[f"w1_{i}"], vec(f"b1_{i}"), vec(f"g1_{i}"),
                      vec(f"be1_{i}"), params[f"w2_{i}"], vec(f"b2_{i}"),
                      vec(f"g_{i}"), vec(f"be_{i}"))
        if i < _L - 1:
            h = _gin_layer(h, agg, *layer_args)
        else:
            out = _final_layer(h, agg, batch2d, *layer_args,
                               params["lin1_w"], vec("lin1_b"), vec("bn1_g"),
                               vec("bn1_b"), params["lin2_w"], vec("lin2_b"))
    return out


# speedup vs baseline: 1.0671x; 1.0671x over previous
import functools

import jax
import jax.numpy as jnp
import numpy as np
from jax import lax
from jax.experimental import pallas as pl
from jax.experimental.pallas import tpu as pltpu
from jax.experimental.pallas import tpu_sc as plsc

_N = 10000
_E = 320000
_D = 128
_H = 128
_B = 64
_OUT = 40
_L = 3

_NC = 2
_NS = 16
_NW = _NC * _NS
_CHUNK = 128
_NCHUNK = 80
_HC = 40
_EPAD = _NW * _NCHUNK * _CHUNK
_NPAD = 10240
_RPT = _NPAD // _NS


_PAD_LANES = np.arange(_EPAD - _E, dtype=np.int32) % 128
_PAD_BLOCK = np.stack([_PAD_LANES * (_N // 128), _N + _PAD_LANES])


def _seg_sum_body(h_hbm, edges_hbm, out_hbm,
                  accum, idx_s, idx_d, rows, sems):
    c = lax.axis_index("c")
    s = lax.axis_index("s")
    wid = c * _NS + s

    @pl.loop(0, _CHUNK)
    def _zero_rows(r):
        @pl.loop(0, _D // 16)
        def _zero_lanes(k):
            rows[0, r, pl.ds(k * 16, 16)] = jnp.zeros((16,), jnp.float32)

    for k in range(_RPT // _CHUNK):
        pltpu.sync_copy(rows.at[0],
                        accum.at[pl.ds(s * _RPT + k * _CHUNK, _CHUNK)])
    plsc.subcore_barrier()

    def start(j, b):
        pltpu.async_copy(h_hbm.at[idx_s.at[j]], rows.at[b], sems.at[b])

    def finish(j, b):
        pltpu.make_async_copy(h_hbm.at[idx_s.at[j]], rows.at[b],
                              sems.at[b]).wait()
        pltpu.sync_copy(rows.at[b], accum.at[idx_d.at[j]], add=True)

    for stage in range(_NCHUNK // _HC):
        pltpu.sync_copy(edges_hbm.at[0].at[wid].at[stage], idx_s)
        pltpu.sync_copy(edges_hbm.at[1].at[wid].at[stage], idx_d)

        start(0, 0)
        start(1, 1)

        @pl.loop(0, _HC - 2, step=2)
        def _edge_loop(k):
            for b in range(2):
                j = k + b
                finish(j, b)
                start(j + 2, b)

        for b in range(2):
            finish(_HC - 2 + b, b)

    plsc.subcore_barrier()
    pltpu.sync_copy(accum.at[pl.ds(s * _RPT, _RPT)],
                    out_hbm.at[c].at[pl.ds(s * _RPT, _RPT)])


_segment_sum_sc = functools.partial(
    pl.kernel,
    out_type=jax.ShapeDtypeStruct((_NC, _NPAD, _D), jnp.float32),
    mesh=plsc.VectorSubcoreMesh(core_axis_name="c", subcore_axis_name="s"),
    scratch_types=[
        pltpu.VMEM_SHARED((_NPAD, _D), jnp.float32),
        pltpu.VMEM((_HC, _CHUNK), jnp.int32),
        pltpu.VMEM((_HC, _CHUNK), jnp.int32),
        pltpu.VMEM((2, _CHUNK, _D), jnp.float32),
        pltpu.SemaphoreType.DMA((2,)),
    ],
)(_seg_sum_body)


def _bn_relu(z, g, b):
    m = jnp.mean(z, axis=0, keepdims=True)
    v = jnp.mean((z - m) ** 2, axis=0, keepdims=True)
    return jnp.maximum((z - m) / jnp.sqrt(v + 1e-5) * g + b, 0.0)


def _dot_bf16(a, b):
    return jnp.dot(a.astype(jnp.bfloat16), b.astype(jnp.bfloat16),
                   preferred_element_type=jnp.float32)


_BLK = 1000
_NB = _N // _BLK


def _phase_a(i, h_ref, agg_ref, eps_ref, w1_ref, b1_ref, z_acc, s1, s2):
    @pl.when(i == 0)
    def _init():
        s1[...] = jnp.zeros_like(s1)
        s2[...] = jnp.zeros_like(s2)

    @pl.when(i < _NB)
    def _stream():
        y = (1.0 + eps_ref[0, 0]) * h_ref[...] + agg_ref[0] + agg_ref[1]
        z = _dot_bf16(y, w1_ref[...]) + b1_ref[...]
        z_acc[pl.ds(i * _BLK, _BLK), :] = z
        s1[...] += jnp.sum(z, axis=0, keepdims=True)
        s2[...] += jnp.sum(z * z, axis=0, keepdims=True)


def _phase_b(g1_ref, be1_ref, w2_ref, b2_ref, g_ref, be_ref, z_acc, s1, s2):
    m = s1[...] / _N
    v = s2[...] / _N - m * m
    z = jnp.maximum((z_acc[...] - m) / jnp.sqrt(v + 1e-5) * g1_ref[...]
                    + be1_ref[...], 0.0)
    a = _dot_bf16(z, w2_ref[...]) + b2_ref[...]
    return _bn_relu(a, g_ref[...], be_ref[...])


def _gin_layer_body(h_ref, agg_ref, eps_ref, w1_ref, b1_ref, g1_ref, be1_ref,
                    w2_ref, b2_ref, g_ref, be_ref, out_ref, z_acc, s1, s2):
    i = pl.program_id(0)
    _phase_a(i, h_ref, agg_ref, eps_ref, w1_ref, b1_ref, z_acc, s1, s2)

    @pl.when(i == _NB)
    def _final():
        out_ref[...] = _phase_b(g1_ref, be1_ref, w2_ref, b2_ref, g_ref,
                                be_ref, z_acc, s1, s2)


def _final_body(h_ref, agg_ref, batch_ref, eps_ref, w1_ref, b1_ref, g1_ref,
                be1_ref, w2_ref, b2_ref, g_ref, be_ref, l1w_ref, l1b_ref,
                bng_ref, bnb_ref, l2w_ref, l2b_ref, out_ref, z_acc, s1, s2):
    i = pl.program_id(0)
    _phase_a(i, h_ref, agg_ref, eps_ref, w1_ref, b1_ref, z_acc, s1, s2)

    @pl.when(i == _NB)
    def _final():
        hl = _phase_b(g1_ref, be1_ref, w2_ref, b2_ref, g_ref, be_ref,
                      z_acc, s1, s2)
        seg_ids = lax.broadcasted_iota(jnp.int32, (_B, _N), 0)
        onehot = (seg_ids == batch_ref[...]).astype(jnp.float32)
        pooled = _dot_bf16(onehot, hl)
        z = jnp.dot(pooled, l1w_ref[...], preferred_element_type=jnp.float32)
        z = _bn_relu(z + l1b_ref[...], bng_ref[...], bnb_ref[...])
        z = jnp.dot(z, l2w_ref[...],
                    preferred_element_type=jnp.float32) + l2b_ref[...]
        zm = z - jnp.max(z, axis=1, keepdims=True)
        out_ref[...] = zm - jnp.log(jnp.sum(jnp.exp(zm), axis=1,
                                            keepdims=True))


def _row_block(i):
    return (jnp.minimum(i, _NB - 1), 0)


_LAYER_SPECS = [
    pl.BlockSpec((_BLK, _D), _row_block),
    pl.BlockSpec((2, _BLK, _D), lambda i: (0,) + _row_block(i)),
    pl.BlockSpec(memory_space=pltpu.SMEM),
] + [pl.BlockSpec(memory_space=pltpu.VMEM)] * 8

_SCRATCH = [
    pltpu.VMEM((_N, 2 * _H), jnp.float32),
    pltpu.VMEM((1, 2 * _H), jnp.float32),
    pltpu.VMEM((1, 2 * _H), jnp.float32),
]


def _gin_layer(h, agg, eps, w1, b1, g1, be1, w2, b2, g, be):
    return pl.pallas_call(
        _gin_layer_body,
        grid=(_NB + 1,),
        out_shape=jax.ShapeDtypeStruct((_N, _H), jnp.float32),
        in_specs=_LAYER_SPECS,
        out_specs=pl.BlockSpec((_N, _H), lambda i: (0, 0)),
        scratch_shapes=_SCRATCH,
    )(h, agg, eps, w1, b1, g1, be1, w2, b2, g, be)


def _final_layer(h, agg, batch2d, eps, w1, b1, g1, be1, w2, b2, g, be,
                 l1w, l1b, bng, bnb, l2w, l2b):
    in_specs = (_LAYER_SPECS[:2]
                + [pl.BlockSpec(memory_space=pltpu.VMEM)]
                + _LAYER_SPECS[2:]
                + [pl.BlockSpec(memory_space=pltpu.VMEM)] * 6)
    return pl.pallas_call(
        _final_body,
        grid=(_NB + 1,),
        out_shape=jax.ShapeDtypeStruct((_B, _OUT), jnp.float32),
        in_specs=in_specs,
        out_specs=pl.BlockSpec((_B, _OUT), lambda i: (0, 0)),
        scratch_shapes=_SCRATCH,
    )(h, agg, batch2d, eps, w1, b1, g1, be1, w2, b2, g, be,
      l1w, l1b, bng, bnb, l2w, l2b)


def kernel(x, edge_index, batch, params):
    edges = jnp.concatenate(
        [edge_index, jnp.asarray(_PAD_BLOCK)], axis=1,
    ).reshape(2, _NW, _NCHUNK // _HC, _HC, _CHUNK)
    batch2d = batch.reshape(1, _N)

    def vec(name):
        p = params[name]
        return p.reshape(1, p.shape[0])

    h = x
    out = None
    for i in range(_L):
        agg = _segment_sum_sc(h, edges)
        eps = params[f"eps_{i}"].reshape(1, 1)
        layer_args = (eps, params[f"w1_{i}"], vec(f"b1_{i}"), vec(f"g1_{i}"),
                      vec(f"be1_{i}"), params[f"w2_{i}"], vec(f"b2_{i}"),
                      vec(f"g_{i}"), vec(f"be_{i}"))
        if i < _L - 1:
            h = _gin_layer(h, agg, *layer_args)
        else:
            out = _final_layer(h, agg, batch2d, *layer_args,
                               params["lin1_w"], vec("lin1_b"), vec("bn1_g"),
                               vec("bn1_b"), params["lin2_w"], vec("lin2_b"))
    return out


# speedup vs baseline: 1.0852x; 1.0170x over previous
import functools

import jax
import jax.numpy as jnp
import numpy as np
from jax import lax
from jax.experimental import pallas as pl
from jax.experimental.pallas import tpu as pltpu
from jax.experimental.pallas import tpu_sc as plsc

_N = 10000
_E = 320000
_D = 128
_H = 128
_B = 64
_OUT = 40
_L = 3

_NC = 2
_NS = 16
_NW = _NC * _NS
_CHUNK = 128
_NCHUNK = 80
_HC = 40
_EPAD = _NW * _NCHUNK * _CHUNK
_NPAD = 10240
_RPT = _NPAD // _NS


_PAD_LANES = np.arange(_EPAD - _E, dtype=np.int32) % 128
_PAD_BLOCK = np.stack([_PAD_LANES * (_N // 128), _N + _PAD_LANES])


def _seg_sum_body(h_hbm, edges_hbm, out_hbm,
                  accum, idx_s, idx_d, rows, sems):
    c = lax.axis_index("c")
    s = lax.axis_index("s")
    wid = c * _NS + s

    @pl.loop(0, _CHUNK)
    def _zero_rows(r):
        @pl.loop(0, _D // 16)
        def _zero_lanes(k):
            rows[0, r, pl.ds(k * 16, 16)] = jnp.zeros((16,), jnp.float32)

    for k in range(_RPT // _CHUNK):
        pltpu.sync_copy(rows.at[0],
                        accum.at[pl.ds(s * _RPT + k * _CHUNK, _CHUNK)])
    plsc.subcore_barrier()

    def start(j, b):
        pltpu.async_copy(h_hbm.at[idx_s.at[j]], rows.at[b], sems.at[b])

    def finish(j, b):
        pltpu.make_async_copy(h_hbm.at[idx_s.at[j]], rows.at[b],
                              sems.at[b]).wait()
        pltpu.sync_copy(rows.at[b], accum.at[idx_d.at[j]], add=True)

    for stage in range(_NCHUNK // _HC):
        pltpu.sync_copy(edges_hbm.at[0].at[wid].at[stage], idx_s)
        pltpu.sync_copy(edges_hbm.at[1].at[wid].at[stage], idx_d)

        start(0, 0)
        start(1, 1)

        @pl.loop(0, _HC - 2, step=2)
        def _edge_loop(k):
            for b in range(2):
                j = k + b
                finish(j, b)
                start(j + 2, b)

        for b in range(2):
            finish(_HC - 2 + b, b)

    plsc.subcore_barrier()
    pltpu.sync_copy(accum.at[pl.ds(s * _RPT, _RPT)],
                    out_hbm.at[c].at[pl.ds(s * _RPT, _RPT)])


_segment_sum_sc = functools.partial(
    pl.kernel,
    out_type=jax.ShapeDtypeStruct((_NC, _NPAD, _D), jnp.float32),
    mesh=plsc.VectorSubcoreMesh(core_axis_name="c", subcore_axis_name="s"),
    scratch_types=[
        pltpu.VMEM_SHARED((_NPAD, _D), jnp.float32),
        pltpu.VMEM((_HC, _CHUNK), jnp.int32),
        pltpu.VMEM((_HC, _CHUNK), jnp.int32),
        pltpu.VMEM((2, _CHUNK, _D), jnp.float32),
        pltpu.SemaphoreType.DMA((2,)),
    ],
)(_seg_sum_body)


def _bn_relu(z, g, b):
    m = jnp.mean(z, axis=0, keepdims=True)
    v = jnp.mean((z - m) ** 2, axis=0, keepdims=True)
    return jnp.maximum((z - m) / jnp.sqrt(v + 1e-5) * g + b, 0.0)


def _dot_bf16(a, b):
    return jnp.dot(a.astype(jnp.bfloat16), b.astype(jnp.bfloat16),
                   preferred_element_type=jnp.float32)


_BLK = 2000
_NB = _N // _BLK


def _phase_a(i, h_ref, agg_ref, eps_ref, w1_ref, b1_ref, z_acc, s1, s2):
    @pl.when(i == 0)
    def _init():
        s1[...] = jnp.zeros_like(s1)
        s2[...] = jnp.zeros_like(s2)

    @pl.when(i < _NB)
    def _stream():
        y = (1.0 + eps_ref[0, 0]) * h_ref[...] + agg_ref[0] + agg_ref[1]
        z = _dot_bf16(y, w1_ref[...]) + b1_ref[...]
        z_acc[pl.ds(i * _BLK, _BLK), :] = z
        s1[...] += jnp.sum(z, axis=0, keepdims=True)
        s2[...] += jnp.sum(z * z, axis=0, keepdims=True)


def _phase_b(g1_ref, be1_ref, w2_ref, b2_ref, g_ref, be_ref, z_acc, s1, s2):
    m = s1[...] / _N
    v = s2[...] / _N - m * m
    z = jnp.maximum((z_acc[...] - m) / jnp.sqrt(v + 1e-5) * g1_ref[...]
                    + be1_ref[...], 0.0)
    a = _dot_bf16(z, w2_ref[...]) + b2_ref[...]
    return _bn_relu(a, g_ref[...], be_ref[...])


def _gin_layer_body(h_ref, agg_ref, eps_ref, w1_ref, b1_ref, g1_ref, be1_ref,
                    w2_ref, b2_ref, g_ref, be_ref, out_ref, z_acc, s1, s2):
    i = pl.program_id(0)
    _phase_a(i, h_ref, agg_ref, eps_ref, w1_ref, b1_ref, z_acc, s1, s2)

    @pl.when(i == _NB)
    def _final():
        out_ref[...] = _phase_b(g1_ref, be1_ref, w2_ref, b2_ref, g_ref,
                                be_ref, z_acc, s1, s2)


def _final_body(h_ref, agg_ref, batch_ref, eps_ref, w1_ref, b1_ref, g1_ref,
                be1_ref, w2_ref, b2_ref, g_ref, be_ref, l1w_ref, l1b_ref,
                bng_ref, bnb_ref, l2w_ref, l2b_ref, out_ref, z_acc, s1, s2):
    i = pl.program_id(0)
    _phase_a(i, h_ref, agg_ref, eps_ref, w1_ref, b1_ref, z_acc, s1, s2)

    @pl.when(i == _NB)
    def _final():
        hl = _phase_b(g1_ref, be1_ref, w2_ref, b2_ref, g_ref, be_ref,
                      z_acc, s1, s2)
        seg_ids = lax.broadcasted_iota(jnp.int32, (_B, _N), 0)
        onehot = (seg_ids == batch_ref[...]).astype(jnp.float32)
        pooled = _dot_bf16(onehot, hl)
        z = jnp.dot(pooled, l1w_ref[...], preferred_element_type=jnp.float32)
        z = _bn_relu(z + l1b_ref[...], bng_ref[...], bnb_ref[...])
        z = jnp.dot(z, l2w_ref[...],
                    preferred_element_type=jnp.float32) + l2b_ref[...]
        zm = z - jnp.max(z, axis=1, keepdims=True)
        out_ref[...] = zm - jnp.log(jnp.sum(jnp.exp(zm), axis=1,
                                            keepdims=True))


def _row_block(i):
    return (jnp.minimum(i, _NB - 1), 0)


_LAYER_SPECS = [
    pl.BlockSpec((_BLK, _D), _row_block),
    pl.BlockSpec((2, _BLK, _D), lambda i: (0,) + _row_block(i)),
    pl.BlockSpec(memory_space=pltpu.SMEM),
] + [pl.BlockSpec(memory_space=pltpu.VMEM)] * 8

_SCRATCH = [
    pltpu.VMEM((_N, 2 * _H), jnp.float32),
    pltpu.VMEM((1, 2 * _H), jnp.float32),
    pltpu.VMEM((1, 2 * _H), jnp.float32),
]


def _gin_layer(h, agg, eps, w1, b1, g1, be1, w2, b2, g, be):
    return pl.pallas_call(
        _gin_layer_body,
        grid=(_NB + 1,),
        out_shape=jax.ShapeDtypeStruct((_N, _H), jnp.float32),
        in_specs=_LAYER_SPECS,
        out_specs=pl.BlockSpec((_N, _H), lambda i: (0, 0)),
        scratch_shapes=_SCRATCH,
    )(h, agg, eps, w1, b1, g1, be1, w2, b2, g, be)


def _final_layer(h, agg, batch2d, eps, w1, b1, g1, be1, w2, b2, g, be,
                 l1w, l1b, bng, bnb, l2w, l2b):
    in_specs = (_LAYER_SPECS[:2]
                + [pl.BlockSpec(memory_space=pltpu.VMEM)]
                + _LAYER_SPECS[2:]
                + [pl.BlockSpec(memory_space=pltpu.VMEM)] * 6)
    return pl.pallas_call(
        _final_body,
        grid=(_NB + 1,),
        out_shape=jax.ShapeDtypeStruct((_B, _OUT), jnp.float32),
        in_specs=in_specs,
        out_specs=pl.BlockSpec((_B, _OUT), lambda i: (0, 0)),
        scratch_shapes=_SCRATCH,
    )(h, agg, batch2d, eps, w1, b1, g1, be1, w2, b2, g, be,
      l1w, l1b, bng, bnb, l2w, l2b)


def kernel(x, edge_index, batch, params):
    edges = jnp.concatenate(
        [edge_index, jnp.asarray(_PAD_BLOCK)], axis=1,
    ).reshape(2, _NW, _NCHUNK // _HC, _HC, _CHUNK)
    batch2d = batch.reshape(1, _N)

    def vec(name):
        p = params[name]
        return p.reshape(1, p.shape[0])

    h = x
    out = None
    for i in range(_L):
        agg = _segment_sum_sc(h, edges)
        eps = params[f"eps_{i}"].reshape(1, 1)
        layer_args = (eps, params[f"w1_{i}"], vec(f"b1_{i}"), vec(f"g1_{i}"),
                      vec(f"be1_{i}"), params[f"w2_{i}"], vec(f"b2_{i}"),
                      vec(f"g_{i}"), vec(f"be_{i}"))
        if i < _L - 1:
            h = _gin_layer(h, agg, *layer_args)
        else:
            out = _final_layer(h, agg, batch2d, *layer_args,
                               params["lin1_w"], vec("lin1_b"), vec("bn1_g"),
                               vec("bn1_b"), params["lin2_w"], vec("lin2_b"))
    return out


# speedup vs baseline: 1.0940x; 1.0081x over previous
import functools

import jax
import jax.numpy as jnp
import numpy as np
from jax import lax
from jax.experimental import pallas as pl
from jax.experimental.pallas import tpu as pltpu
from jax.experimental.pallas import tpu_sc as plsc

_N = 10000
_E = 320000
_D = 128
_H = 128
_B = 64
_OUT = 40
_L = 3

_NC = 2
_NS = 16
_NW = _NC * _NS
_CHUNK = 128
_NCHUNK = 80
_HC = 40
_EPAD = _NW * _NCHUNK * _CHUNK
_NPAD = 10240
_RPT = _NPAD // _NS


_PAD_LANES = np.arange(_EPAD - _E, dtype=np.int32) % 128
_PAD_BLOCK = np.stack([_PAD_LANES * (_N // 128), _N + _PAD_LANES])


def _seg_sum_body(h_hbm, edges_hbm, out_hbm,
                  accum, idx_s, idx_d, rows, sems):
    c = lax.axis_index("c")
    s = lax.axis_index("s")
    wid = c * _NS + s

    @pl.loop(0, _CHUNK)
    def _zero_rows(r):
        @pl.loop(0, _D // 16)
        def _zero_lanes(k):
            rows[0, r, pl.ds(k * 16, 16)] = jnp.zeros((16,), jnp.float32)

    for k in range(_RPT // _CHUNK):
        pltpu.sync_copy(rows.at[0],
                        accum.at[pl.ds(s * _RPT + k * _CHUNK, _CHUNK)])
    plsc.subcore_barrier()

    def start(j, b):
        pltpu.async_copy(h_hbm.at[idx_s.at[j]], rows.at[b], sems.at[b])

    def finish(j, b):
        pltpu.make_async_copy(h_hbm.at[idx_s.at[j]], rows.at[b],
                              sems.at[b]).wait()
        pltpu.sync_copy(rows.at[b], accum.at[idx_d.at[j]], add=True)

    for stage in range(_NCHUNK // _HC):
        pltpu.sync_copy(edges_hbm.at[0].at[wid].at[stage], idx_s)
        pltpu.sync_copy(edges_hbm.at[1].at[wid].at[stage], idx_d)

        start(0, 0)
        start(1, 1)

        @pl.loop(0, _HC - 2, step=2)
        def _edge_loop(k):
            for b in range(2):
                j = k + b
                finish(j, b)
                start(j + 2, b)

        for b in range(2):
            finish(_HC - 2 + b, b)

    plsc.subcore_barrier()
    pltpu.sync_copy(accum.at[pl.ds(s * _RPT, _RPT)],
                    out_hbm.at[c].at[pl.ds(s * _RPT, _RPT)])


_segment_sum_sc = functools.partial(
    pl.kernel,
    out_type=jax.ShapeDtypeStruct((_NC, _NPAD, _D), jnp.float32),
    mesh=plsc.VectorSubcoreMesh(core_axis_name="c", subcore_axis_name="s"),
    scratch_types=[
        pltpu.VMEM_SHARED((_NPAD, _D), jnp.float32),
        pltpu.VMEM((_HC, _CHUNK), jnp.int32),
        pltpu.VMEM((_HC, _CHUNK), jnp.int32),
        pltpu.VMEM((2, _CHUNK, _D), jnp.float32),
        pltpu.SemaphoreType.DMA((2,)),
    ],
)(_seg_sum_body)


def _bn_relu(z, g, b):
    m = jnp.mean(z, axis=0, keepdims=True)
    v = jnp.mean((z - m) ** 2, axis=0, keepdims=True)
    return jnp.maximum((z - m) / jnp.sqrt(v + 1e-5) * g + b, 0.0)


def _dot_bf16(a, b):
    return jnp.dot(a.astype(jnp.bfloat16), b.astype(jnp.bfloat16),
                   preferred_element_type=jnp.float32)


_BLK = 5000
_NB = _N // _BLK


def _phase_a(i, h_ref, agg_ref, eps_ref, w1_ref, b1_ref, z_acc, s1, s2):
    @pl.when(i == 0)
    def _init():
        s1[...] = jnp.zeros_like(s1)
        s2[...] = jnp.zeros_like(s2)

    @pl.when(i < _NB)
    def _stream():
        y = (1.0 + eps_ref[0, 0]) * h_ref[...] + agg_ref[0] + agg_ref[1]
        z = _dot_bf16(y, w1_ref[...]) + b1_ref[...]
        z_acc[pl.ds(i * _BLK, _BLK), :] = z
        s1[...] += jnp.sum(z, axis=0, keepdims=True)
        s2[...] += jnp.sum(z * z, axis=0, keepdims=True)


def _phase_b(g1_ref, be1_ref, w2_ref, b2_ref, g_ref, be_ref, z_acc, s1, s2):
    m = s1[...] / _N
    v = s2[...] / _N - m * m
    z = jnp.maximum((z_acc[...] - m) / jnp.sqrt(v + 1e-5) * g1_ref[...]
                    + be1_ref[...], 0.0)
    a = _dot_bf16(z, w2_ref[...]) + b2_ref[...]
    return _bn_relu(a, g_ref[...], be_ref[...])


def _gin_layer_body(h_ref, agg_ref, eps_ref, w1_ref, b1_ref, g1_ref, be1_ref,
                    w2_ref, b2_ref, g_ref, be_ref, out_ref, z_acc, s1, s2):
    i = pl.program_id(0)
    _phase_a(i, h_ref, agg_ref, eps_ref, w1_ref, b1_ref, z_acc, s1, s2)

    @pl.when(i == _NB)
    def _final():
        out_ref[...] = _phase_b(g1_ref, be1_ref, w2_ref, b2_ref, g_ref,
                                be_ref, z_acc, s1, s2)


def _final_body(h_ref, agg_ref, batch_ref, eps_ref, w1_ref, b1_ref, g1_ref,
                be1_ref, w2_ref, b2_ref, g_ref, be_ref, l1w_ref, l1b_ref,
                bng_ref, bnb_ref, l2w_ref, l2b_ref, out_ref, z_acc, s1, s2):
    i = pl.program_id(0)
    _phase_a(i, h_ref, agg_ref, eps_ref, w1_ref, b1_ref, z_acc, s1, s2)

    @pl.when(i == _NB)
    def _final():
        hl = _phase_b(g1_ref, be1_ref, w2_ref, b2_ref, g_ref, be_ref,
                      z_acc, s1, s2)
        seg_ids = lax.broadcasted_iota(jnp.int32, (_B, _N), 0)
        onehot = (seg_ids == batch_ref[...]).astype(jnp.float32)
        pooled = _dot_bf16(onehot, hl)
        z = jnp.dot(pooled, l1w_ref[...], preferred_element_type=jnp.float32)
        z = _bn_relu(z + l1b_ref[...], bng_ref[...], bnb_ref[...])
        z = jnp.dot(z, l2w_ref[...],
                    preferred_element_type=jnp.float32) + l2b_ref[...]
        zm = z - jnp.max(z, axis=1, keepdims=True)
        out_ref[...] = zm - jnp.log(jnp.sum(jnp.exp(zm), axis=1,
                                            keepdims=True))


def _row_block(i):
    return (jnp.minimum(i, _NB - 1), 0)


_LAYER_SPECS = [
    pl.BlockSpec((_BLK, _D), _row_block),
    pl.BlockSpec((2, _BLK, _D), lambda i: (0,) + _row_block(i)),
    pl.BlockSpec(memory_space=pltpu.SMEM),
] + [pl.BlockSpec(memory_space=pltpu.VMEM)] * 8

_SCRATCH = [
    pltpu.VMEM((_N, 2 * _H), jnp.float32),
    pltpu.VMEM((1, 2 * _H), jnp.float32),
    pltpu.VMEM((1, 2 * _H), jnp.float32),
]


def _gin_layer(h, agg, eps, w1, b1, g1, be1, w2, b2, g, be):
    return pl.pallas_call(
        _gin_layer_body,
        grid=(_NB + 1,),
        out_shape=jax.ShapeDtypeStruct((_N, _H), jnp.float32),
        in_specs=_LAYER_SPECS,
        out_specs=pl.BlockSpec((_N, _H), lambda i: (0, 0)),
        scratch_shapes=_SCRATCH,
    )(h, agg, eps, w1, b1, g1, be1, w2, b2, g, be)


def _final_layer(h, agg, batch2d, eps, w1, b1, g1, be1, w2, b2, g, be,
                 l1w, l1b, bng, bnb, l2w, l2b):
    in_specs = (_LAYER_SPECS[:2]
                + [pl.BlockSpec(memory_space=pltpu.VMEM)]
                + _LAYER_SPECS[2:]
                + [pl.BlockSpec(memory_space=pltpu.VMEM)] * 6)
    return pl.pallas_call(
        _final_body,
        grid=(_NB + 1,),
        out_shape=jax.ShapeDtypeStruct((_B, _OUT), jnp.float32),
        in_specs=in_specs,
        out_specs=pl.BlockSpec((_B, _OUT), lambda i: (0, 0)),
        scratch_shapes=_SCRATCH,
    )(h, agg, batch2d, eps, w1, b1, g1, be1, w2, b2, g, be,
      l1w, l1b, bng, bnb, l2w, l2b)


def kernel(x, edge_index, batch, params):
    edges = jnp.concatenate(
        [edge_index, jnp.asarray(_PAD_BLOCK)], axis=1,
    ).reshape(2, _NW, _NCHUNK // _HC, _HC, _CHUNK)
    batch2d = batch.reshape(1, _N)

    def vec(name):
        p = params[name]
        return p.reshape(1, p.shape[0])

    h = x
    out = None
    for i in range(_L):
        agg = _segment_sum_sc(h, edges)
        eps = params[f"eps_{i}"].reshape(1, 1)
        layer_args = (eps, params[f"w1_{i}"], vec(f"b1_{i}"), vec(f"g1_{i}"),
                      vec(f"be1_{i}"), params[f"w2_{i}"], vec(f"b2_{i}"),
                      vec(f"g_{i}"), vec(f"be_{i}"))
        if i < _L - 1:
            h = _gin_layer(h, agg, *layer_args)
        else:
            out = _final_layer(h, agg, batch2d, *layer_args,
                               params["lin1_w"], vec("lin1_b"), vec("bn1_g"),
                               vec("bn1_b"), params["lin2_w"], vec("lin2_b"))
    return out
